# baseline (speedup 1.0000x reference)
import functools

import jax
import jax.numpy as jnp
from jax import lax
from jax.experimental import pallas as pl
from jax.experimental.pallas import tpu as pltpu


def _pick_block(n, candidates):
    for c in candidates:
        if n % c == 0:
            return c
    return n


def _leaky(x):
    return jnp.where(x >= 0, x, 0.2 * x)


def _rownorm(h):
    n2 = jnp.sum(h * h, axis=1, keepdims=True)
    inv = lax.rsqrt(jnp.where(n2 > 0, n2, 1.0))
    return jnp.where(n2 > 0, h * inv, 0.0)


def _gru_step(xt_bf, h, wi, bi, wh, bh, H):
    gi = jnp.dot(xt_bf, wi, preferred_element_type=jnp.float32) + bi
    gh = jnp.dot(h.astype(jnp.bfloat16), wh,
                 preferred_element_type=jnp.float32) + bh
    r = jax.nn.sigmoid(gi[:, :H] + gh[:, :H])
    z = jax.nn.sigmoid(gi[:, H:2 * H] + gh[:, H:2 * H])
    nn_ = jnp.tanh(gi[:, 2 * H:] + r * gh[:, 2 * H:])
    return (1.0 - z) * nn_ + z * h


def _gru_body(x_ref, wi0, bi0, wh0, bh0, wi1, bi1, wh1, bh1,
              h_out, n_out, h1_s, h2_s):
    t = pl.program_id(1)
    T = pl.num_programs(1)
    H = wh0.shape[0]

    @pl.when(t == 0)
    def _():
        h1_s[...] = jnp.zeros_like(h1_s)
        h2_s[...] = jnp.zeros_like(h2_s)

    xt = x_ref[0]
    h1n = _gru_step(xt, h1_s[...], wi0[...], bi0[...], wh0[...], bh0[...], H)
    h1_s[...] = h1n
    h2n = _gru_step(h1n.astype(jnp.bfloat16), h2_s[...], wi1[...], bi1[...],
                    wh1[...], bh1[...], H)
    h2_s[...] = h2n

    @pl.when(t == T - 1)
    def _():
        h_out[...] = h2n
        s = jnp.sum(h2n * h2n, axis=1, keepdims=True)
        n_out[...] = jnp.where(s == 0, 0.0,
                               jnp.sqrt(jnp.where(s == 0, 1.0, s)))


def _run_gru(x, p):
    N, T, D = x.shape
    H = p['W_hh0'].shape[1]
    Bg = _pick_block(N, [2000, 1000, 400, 200, 80, 40, 16, 8])
    G = N // Bg
    xt = jnp.swapaxes(x, 0, 1).astype(jnp.bfloat16)
    f32 = jnp.float32
    bf16 = jnp.bfloat16

    def layer(wi, wh, bi, bh):
        return (wi.T.astype(bf16), bi.reshape(1, -1),
                wh.T.astype(bf16), bh.reshape(1, -1))

    l0 = layer(p['W_ih0'], p['W_hh0'], p['b_ih0'], p['b_hh0'])
    l1 = layer(p['W_ih1'], p['W_hh1'], p['b_ih1'], p['b_hh1'])

    wspec = pl.BlockSpec(None, lambda g, t: (0, 0))
    h_out, n_out = pl.pallas_call(
        _gru_body,
        grid=(G, T),
        in_specs=[
            pl.BlockSpec((1, Bg, D), lambda g, t: (t, g, 0)),
        ] + [wspec] * 8,
        out_specs=[
            pl.BlockSpec((Bg, H), lambda g, t: (g, 0)),
            pl.BlockSpec((Bg, 1), lambda g, t: (g, 0)),
        ],
        out_shape=[
            jax.ShapeDtypeStruct((N, H), f32),
            jax.ShapeDtypeStruct((N, 1), f32),
        ],
        scratch_shapes=[
            pltpu.VMEM((Bg, H), f32),
            pltpu.VMEM((Bg, H), f32),
        ],
    )(xt, *l0, *l1)
    return h_out, n_out


def _topk_body(h_blk_bf, h_full_bf, h_blk, n_blk, nt_full, pre_ref, cs_ref,
               *, K, B, N):
    i = pl.program_id(0)
    NP = h_full_bf.shape[0]

    xy = lax.dot_general(h_blk_bf[...], h_full_bf[...],
                         (((1,), (1,)), ((), ())),
                         preferred_element_type=jnp.float32)
    den = n_blk[...] * nt_full[...]
    S = jnp.where(den == 0, 0.0, xy / jnp.where(den == 0, 1.0, den))
    rows = i * B + lax.broadcasted_iota(jnp.int32, (B, NP), 0)
    cols = lax.broadcasted_iota(jnp.int32, (B, NP), 1)
    S = jnp.where(rows == cols, 0.0, S)
    A = jnp.abs(S)
    A = jnp.where(cols >= N, -1.0, A)

    Q = NP // 4
    a, b = A[:, 0 * Q:1 * Q], A[:, 1 * Q:2 * Q]
    c, d = A[:, 2 * Q:3 * Q], A[:, 3 * Q:4 * Q]
    ab_hi, ab_lo = jnp.maximum(a, b), jnp.minimum(a, b)
    cd_hi, cd_lo = jnp.maximum(c, d), jnp.minimum(c, d)
    q1 = jnp.maximum(ab_hi, cd_hi)
    t_ = jnp.minimum(ab_hi, cd_hi)
    q4 = jnp.minimum(ab_lo, cd_lo)
    u_ = jnp.maximum(ab_lo, cd_lo)
    q2 = jnp.maximum(t_, u_)
    q3 = jnp.minimum(t_, u_)

    def body(_, carry):
        w1, w2, w3, w4, _t = carry
        m = jnp.max(w1, axis=1, keepdims=True)
        sel = w1 >= m
        return (jnp.where(sel, w2, w1), jnp.where(sel, w3, w2),
                jnp.where(sel, w4, w3), jnp.where(sel, -1.0, w4), m)

    thr0 = jnp.zeros((B, 1), jnp.float32)
    *_, thr = lax.fori_loop(0, K, body, (q1, q2, q3, q4, thr0))
    TS = jnp.where(A >= thr, S, 0.0)

    @pl.when(i == 0)
    def _():
        pre_ref[...] = jnp.zeros_like(pre_ref)
        cs_ref[...] = jnp.zeros_like(cs_ref)

    pre_ref[...] += lax.dot_general(TS, h_blk[...], (((0,), (0,)), ((), ())))
    cs_ref[...] += lax.dot_general(TS, jnp.ones((B, 1), jnp.float32),
                                   (((0,), (0,)), ((), ())))


def _round_up(n, m):
    return ((n + m - 1) // m) * m


def _run_topk_accum(h, n, K):
    N, H = h.shape
    B = _pick_block(N, [200, 80, 40, 16, 8])
    NP = _round_up(N, 4096) if N >= 4096 else _round_up(N, 64)
    h_bf = h.astype(jnp.bfloat16)
    h_bf_pad = jnp.pad(h_bf, ((0, NP - N), (0, 0)))
    nt_pad = jnp.pad(n[:, 0], (0, NP - N)).reshape(1, NP)
    f32 = jnp.float32
    pre, cs = pl.pallas_call(
        functools.partial(_topk_body, K=K, B=B, N=N),
        grid=(N // B,),
        in_specs=[
            pl.BlockSpec((B, H), lambda i: (i, 0)),
            pl.BlockSpec(None, lambda i: (0, 0)),
            pl.BlockSpec((B, H), lambda i: (i, 0)),
            pl.BlockSpec((B, 1), lambda i: (i, 0)),
            pl.BlockSpec(None, lambda i: (0, 0)),
        ],
        out_specs=[
            pl.BlockSpec(None, lambda i: (0, 0)),
            pl.BlockSpec(None, lambda i: (0, 0)),
        ],
        out_shape=[
            jax.ShapeDtypeStruct((NP, H), f32),
            jax.ShapeDtypeStruct((NP, 1), f32),
        ],
    )(h_bf, h_bf_pad, h, n, nt_pad)
    return pre[:N], cs[:N]


def _att_body(pre_ref, cs_ref, hbf_ref, h_ref, n_ref, wc, bc,
              whs, bhs, wback, bback, wfore, bfore, windi, bindi,
              wout, bout, pred_ref, concept_s, cbf_s, nc_s, acc_s, *, Bc):
    i = pl.program_id(0)
    G = pl.num_programs(0)

    @pl.when(i == 0)
    def _():
        d = jnp.where((cs_ref[...] != 0) & (n_ref[...] > 0), 1.0, 0.0)
        pre_c = pre_ref[...] + d * h_ref[...]
        valid = jnp.where(jnp.sum(pre_c, axis=1, keepdims=True) != 0, 1.0, 0.0)
        concept = _leaky(jnp.dot(pre_c, wc[...]) + bc[...]) * valid
        concept_s[...] = concept
        cbf_s[...] = concept.astype(jnp.bfloat16)
        s2 = jnp.sum(concept * concept, axis=1, keepdims=True)
        nc_s[...] = jnp.where(s2 == 0, 0.0,
                              jnp.sqrt(jnp.where(s2 == 0, 1.0, s2)))
        acc_s[...] = jnp.zeros_like(acc_s)

    cbf_j = cbf_s[pl.ds(i * Bc, Bc), :]
    xy = lax.dot_general(hbf_ref[...], cbf_j, (((1,), (1,)), ((), ())),
                         preferred_element_type=jnp.float32)
    den = lax.dot_general(n_ref[...], nc_s[pl.ds(i * Bc, Bc), :],
                          (((1,), (1,)), ((), ())),
                          preferred_element_type=jnp.float32)
    C = jnp.where(den == 0, 0.0, xy / jnp.where(den == 0, 1.0, den))
    e = jnp.exp(C - 1.0)
    s = jnp.sum(e, axis=0, keepdims=True)
    att = e / s
    acc_s[...] += jnp.dot(att, concept_s[pl.ds(i * Bc, Bc), :])

    @pl.when(i == G - 1)
    def _():
        hsh = acc_s[...]
        hs = _leaky(jnp.dot(hsh, whs[...]) + bhs[...])
        hb = _leaky(jnp.dot(hs, wback[...]) + bback[...])
        ofs = _leaky(jnp.dot(hs, wfore[...]) + bfore[...])
        indi = h_ref[...] - hb
        oin = _leaky(jnp.dot(indi, windi[...]) + bindi[...])
        pred_ref[...] = jnp.dot(ofs + oin, wout[...]) + bout[...]


def _run_att_head(pre, cs, h, n, p):
    N, H = h.shape
    Bc = _pick_block(N, [400, 200, 80, 40, 16, 8])
    f32 = jnp.float32
    spec = pl.BlockSpec(None, lambda i: (0, 0))
    pred = pl.pallas_call(
        functools.partial(_att_body, Bc=Bc),
        grid=(N // Bc,),
        in_specs=[spec] * 17,
        out_specs=pl.BlockSpec(None, lambda i: (0, 0)),
        out_shape=jax.ShapeDtypeStruct((N, 1), f32),
        scratch_shapes=[
            pltpu.VMEM((N, H), f32),
            pltpu.VMEM((N, H), jnp.bfloat16),
            pltpu.VMEM((N, 1), f32),
            pltpu.VMEM((N, H), f32),
        ],
    )(
        pre, cs, h.astype(jnp.bfloat16), h, n,
        p['W_c'].T, p['b_c'].reshape(1, -1),
        p['W_hs'].T, p['b_hs'].reshape(1, -1),
        p['W_back'].T, p['b_back'].reshape(1, -1),
        p['W_fore'].T, p['b_fore'].reshape(1, -1),
        p['W_indi'].T, p['b_indi'].reshape(1, -1),
        p['W_out'].T, p['b_out'].reshape(1, -1),
    )
    return pred


K_TOP = 20


def kernel(x, params):
    h, n = _run_gru(x, params)
    pre, cs = _run_topk_accum(h, n, K_TOP)
    pred = _run_att_head(pre, cs, h, n, params)
    return pred[:, 0]
